# baseline (speedup 1.0000x reference)
import functools

import jax
import jax.numpy as jnp
from jax import lax
from jax.experimental import pallas as pl
from jax.experimental.pallas import tpu as pltpu
from jax.experimental.pallas import tpu_sc as plsc

N = 10000
E = 320000
D = 128
H = 128

BLK = 512
NPAD = 10240
GB = 1024

NC = 2
NS = 16
NW = NC * NS
CH = 128
NB = 2
IG = 8
NCHUNK = 80
EPT = NCHUNK * CH
EPAD = NW * EPT
ROWS_PT = NPAD // NS


def _make_sc_scatter(width, with_gather):
    mesh = plsc.VectorSubcoreMesh(core_axis_name="c", subcore_axis_name="s",
                                  num_cores=NC, num_subcores=NS)

    if with_gather:
        scratch = ([pltpu.VMEM((IG, CH), jnp.int32)]
                   + [pltpu.VMEM((IG, CH), jnp.int32)]
                   + [pltpu.VMEM((CH, width), jnp.float32)] * NB
                   + [pltpu.VMEM_SHARED((NPAD, width), jnp.float32)]
                   + [pltpu.SemaphoreType.DMA]
                   + [pltpu.SemaphoreType.DMA] * NB)
    else:
        scratch = [
            pltpu.VMEM((NCHUNK, CH), jnp.int32),
            pltpu.VMEM((CH, width), jnp.float32),
            pltpu.VMEM_SHARED((NPAD, width), jnp.float32),
            pltpu.SemaphoreType.DMA,
        ]

    @functools.partial(
        pl.kernel,
        out_type=jax.ShapeDtypeStruct((NC, NPAD, width), jnp.float32),
        mesh=mesh,
        scratch_types=scratch,
    )
    def k(*refs):
        if with_gather:
            (rows_hbm, src_hbm, dst_hbm, zeros_hbm, out_hbm,
             srcb, dstb) = refs[:7]
            rowbufs = refs[7:7 + NB]
            acc, sem = refs[7 + NB:9 + NB]
            gsems = refs[9 + NB:]
        else:
            (ones_hbm, dst_hbm, zeros_hbm, out_hbm,
             dstb, rowb0, acc, sem) = refs
        c = lax.axis_index("c")
        s = lax.axis_index("s")
        wid = c * NS + s

        r0 = s * ROWS_PT
        pltpu.sync_copy(zeros_hbm.at[pl.ds(r0, ROWS_PT)],
                        acc.at[pl.ds(r0, ROWS_PT)])
        if not with_gather:
            pltpu.sync_copy(dst_hbm.at[pl.ds(wid * NCHUNK, NCHUNK)], dstb)
            pltpu.sync_copy(ones_hbm, rowb0)
        plsc.subcore_barrier()

        if with_gather:
            def group(g, _):
                grow = wid * NCHUNK + g * IG
                pltpu.sync_copy(src_hbm.at[pl.ds(grow, IG)], srcb)
                pltpu.sync_copy(dst_hbm.at[pl.ds(grow, IG)], dstb)
                for b in range(NB):
                    pltpu.async_copy(rows_hbm.at[srcb.at[b]], rowbufs[b],
                                     gsems[b])
                for j in range(IG):
                    b = j % NB
                    pltpu.make_async_copy(rows_hbm.at[srcb.at[j]],
                                          rowbufs[b], gsems[b]).wait()
                    pltpu.sync_copy(rowbufs[b], acc.at[dstb.at[j]], add=True)
                    if j + NB < IG:
                        pltpu.async_copy(rows_hbm.at[srcb.at[j + NB]],
                                         rowbufs[b], gsems[b])
                return 0

            lax.fori_loop(0, NCHUNK // IG, group, 0)
        else:
            def body(j, _):
                pltpu.sync_copy(rowb0, acc.at[dstb.at[j]], add=True)
                return 0

            lax.fori_loop(0, NCHUNK, body, 0)

        plsc.subcore_barrier()
        pltpu.sync_copy(acc.at[pl.ds(r0, ROWS_PT)],
                        out_hbm.at[c, pl.ds(r0, ROWS_PT)])

    return k


_sc_cache = {}


def _sc_deg(*args):
    if "deg" not in _sc_cache:
        _sc_cache["deg"] = _make_sc_scatter(H, with_gather=False)
    return _sc_cache["deg"](*args)


def _sc_rows(*args):
    if "rows" not in _sc_cache:
        _sc_cache["rows"] = _make_sc_scatter(H, with_gather=True)
    return _sc_cache["rows"](*args)


def _dinv(deg2_blk):
    deg = deg2_blk[0, :, 0:1] + deg2_blk[1, :, 0:1] + 1.0
    return lax.rsqrt(deg)


def _enc_pre1_body(x_ref, wencT_ref, benc_ref, wg1T_ref, deg2_ref, out_ref):
    h0 = jnp.dot(x_ref[...], wencT_ref[...],
                 preferred_element_type=jnp.float32) + benc_ref[...]
    dinv = _dinv(deg2_ref[...])
    out_ref[...] = jnp.dot(h0, wg1T_ref[...],
                           preferred_element_type=jnp.float32) * dinv


def _mid_body(s2_ref, hw_ref, deg2_ref, b_ref, wT_ref, out_ref):
    dinv = _dinv(deg2_ref[...])
    tot = s2_ref[0] + s2_ref[1] + hw_ref[...]
    h1 = jax.nn.relu(dinv * tot + b_ref[...])
    out_ref[...] = jnp.dot(h1, wT_ref[...],
                           preferred_element_type=jnp.float32) * dinv


def _post_body(s2_ref, hw_ref, deg2_ref, b_ref, wihT_ref, bih_ref, out_ref):
    dinv = _dinv(deg2_ref[...])
    tot = s2_ref[0] + s2_ref[1] + hw_ref[...]
    h2 = jax.nn.relu(dinv * tot + b_ref[...])
    out_ref[...] = jnp.dot(h2, wihT_ref[...],
                           preferred_element_type=jnp.float32) + bih_ref[...]


def _gru_body(gi_ref, whhT_ref, bhh_ref, wclsT_ref, bcls_ref,
              ys_ref, logits_ref, carry_ref):
    @pl.when(pl.program_id(0) == 0)
    def _():
        carry_ref[...] = jnp.zeros_like(carry_ref)

    def step(j, hprev):
        gh = jnp.dot(hprev, whhT_ref[...],
                     preferred_element_type=jnp.float32) + bhh_ref[...]
        gi = gi_ref[pl.ds(j, 1), :]
        r = jax.nn.sigmoid(gi[:, 0:H] + gh[:, 0:H])
        z = jax.nn.sigmoid(gi[:, H:2 * H] + gh[:, H:2 * H])
        n = jnp.tanh(gi[:, 2 * H:3 * H] + r * gh[:, 2 * H:3 * H])
        h = (1.0 - z) * n + z * hprev
        ys_ref[pl.ds(j, 1), :] = h
        return h

    h_out = lax.fori_loop(0, GB, step, carry_ref[0:1, :])
    carry_ref[0:1, :] = h_out
    logits_ref[...] = jnp.dot(ys_ref[...], wclsT_ref[...],
                              preferred_element_type=jnp.float32) + bcls_ref[...]


def _row_spec(width):
    return pl.BlockSpec((BLK, width), lambda i: (i, 0))


def _full_spec(shape):
    return pl.BlockSpec(shape, lambda i: tuple(0 for _ in shape))


def _deg_spec():
    return pl.BlockSpec((2, BLK, H), lambda i: (0, i, 0))


def _tc_rowwise(body, in_specs, out_width, num_outs=1):
    out_shape = jax.ShapeDtypeStruct((NPAD, out_width), jnp.float32)
    return pl.pallas_call(
        body,
        grid=(NPAD // BLK,),
        in_specs=in_specs,
        out_specs=_row_spec(out_width),
        out_shape=out_shape,
    )


def kernel(x, edge_index, timestamps, W_enc, b_enc, W_g1, b_g1, W_g2, b_g2,
           W_ih, W_hh, b_ih, b_hh, W_cls, b_cls):
    del timestamps
    src = edge_index[0]
    dst = edge_index[1]

    xp = jnp.zeros((NPAD, D), jnp.float32).at[:N].set(x)

    npadding = EPAD - E
    pad_idx = jnp.arange(npadding, dtype=jnp.int32)
    srcp = jnp.concatenate(
        [src, pad_idx % N]).reshape(NW * NCHUNK, CH)
    dstp = jnp.concatenate(
        [dst, N + pad_idx % (NPAD - N)]).reshape(NW * NCHUNK, CH)
    z128 = jnp.zeros((NPAD, H), jnp.float32)
    ones128 = jnp.ones((CH, H), jnp.float32)

    deg2 = _sc_deg(ones128, dstp, z128)

    hw1 = _tc_rowwise(
        _enc_pre1_body,
        [_row_spec(D), _full_spec((D, H)), _full_spec((1, H)),
         _full_spec((H, H)), _deg_spec()],
        H,
    )(xp, W_enc.T, b_enc[None, :], W_g1.T, deg2)

    s1_2 = _sc_rows(hw1, srcp, dstp, z128)

    hw2 = _tc_rowwise(
        _mid_body,
        [pl.BlockSpec((2, BLK, H), lambda i: (0, i, 0)), _row_spec(H),
         _deg_spec(), _full_spec((1, H)), _full_spec((H, H))],
        H,
    )(s1_2, hw1, deg2, b_g1[None, :], W_g2.T)

    s2_2 = _sc_rows(hw2, srcp, dstp, z128)

    gi = _tc_rowwise(
        _post_body,
        [pl.BlockSpec((2, BLK, H), lambda i: (0, i, 0)), _row_spec(H),
         _deg_spec(), _full_spec((1, H)), _full_spec((H, 3 * H)),
         _full_spec((1, 3 * H))],
        3 * H,
    )(s2_2, hw2, deg2, b_g2[None, :], W_ih.T, b_ih[None, :])

    wcls_pad = jnp.zeros((H, 128), jnp.float32).at[:, :2].set(W_cls.T)
    bcls_pad = jnp.zeros((1, 128), jnp.float32).at[0, :2].set(b_cls)
    ys, logits_pad = pl.pallas_call(
        _gru_body,
        grid=(NPAD // GB,),
        in_specs=[pl.BlockSpec((GB, 3 * H), lambda i: (i, 0)),
                  _full_spec((H, 3 * H)), _full_spec((1, 3 * H)),
                  _full_spec((H, 128)), _full_spec((1, 128))],
        out_specs=[pl.BlockSpec((GB, H), lambda i: (i, 0)),
                   pl.BlockSpec((GB, 128), lambda i: (i, 0))],
        out_shape=[jax.ShapeDtypeStruct((NPAD, H), jnp.float32),
                   jax.ShapeDtypeStruct((NPAD, 128), jnp.float32)],
        scratch_shapes=[pltpu.VMEM((8, H), jnp.float32)],
    )(gi, W_hh.T, b_hh[None, :], wcls_pad, bcls_pad)

    return (logits_pad[:N, :2], ys[:N])


# speedup vs baseline: 1.0503x; 1.0503x over previous
import functools

import jax
import jax.numpy as jnp
from jax import lax
from jax.experimental import pallas as pl
from jax.experimental.pallas import tpu as pltpu
from jax.experimental.pallas import tpu_sc as plsc

N = 10000
E = 320000
D = 128
H = 128

BLK = 512
NPAD = 10240
GB = 2048

NC = 2
NS = 16
NW = NC * NS
CH = 128
NB = 2
IG = 8
NCHUNK = 80
EPT = NCHUNK * CH
EPAD = NW * EPT
ROWS_PT = NPAD // NS


def _make_sc_scatter(width, with_gather):
    mesh = plsc.VectorSubcoreMesh(core_axis_name="c", subcore_axis_name="s",
                                  num_cores=NC, num_subcores=NS)

    if with_gather:
        scratch = ([pltpu.VMEM((IG, CH), jnp.int32)]
                   + [pltpu.VMEM((IG, CH), jnp.int32)]
                   + [pltpu.VMEM((CH, width), jnp.float32)] * NB
                   + [pltpu.VMEM_SHARED((NPAD, width), jnp.float32)]
                   + [pltpu.SemaphoreType.DMA]
                   + [pltpu.SemaphoreType.DMA] * NB)
    else:
        scratch = [
            pltpu.VMEM((NCHUNK, CH), jnp.int32),
            pltpu.VMEM((CH, width), jnp.float32),
            pltpu.VMEM_SHARED((NPAD, width), jnp.float32),
            pltpu.SemaphoreType.DMA,
        ]

    @functools.partial(
        pl.kernel,
        out_type=jax.ShapeDtypeStruct((NC, NPAD, width), jnp.float32),
        mesh=mesh,
        scratch_types=scratch,
    )
    def k(*refs):
        if with_gather:
            (rows_hbm, src_hbm, dst_hbm, zeros_hbm, out_hbm,
             srcb, dstb) = refs[:7]
            rowbufs = refs[7:7 + NB]
            acc, sem = refs[7 + NB:9 + NB]
            gsems = refs[9 + NB:]
        else:
            (ones_hbm, dst_hbm, zeros_hbm, out_hbm,
             dstb, rowb0, acc, sem) = refs
        c = lax.axis_index("c")
        s = lax.axis_index("s")
        wid = c * NS + s

        r0 = s * ROWS_PT
        pltpu.sync_copy(zeros_hbm.at[pl.ds(r0, ROWS_PT)],
                        acc.at[pl.ds(r0, ROWS_PT)])
        if not with_gather:
            pltpu.sync_copy(dst_hbm.at[pl.ds(wid * NCHUNK, NCHUNK)], dstb)
            pltpu.sync_copy(ones_hbm, rowb0)
        plsc.subcore_barrier()

        if with_gather:
            def group(g, _):
                grow = wid * NCHUNK + g * IG
                pltpu.sync_copy(src_hbm.at[pl.ds(grow, IG)], srcb)
                pltpu.sync_copy(dst_hbm.at[pl.ds(grow, IG)], dstb)
                for b in range(NB):
                    pltpu.async_copy(rows_hbm.at[srcb.at[b]], rowbufs[b],
                                     gsems[b])
                for j in range(IG):
                    b = j % NB
                    pltpu.make_async_copy(rows_hbm.at[srcb.at[j]],
                                          rowbufs[b], gsems[b]).wait()
                    pltpu.sync_copy(rowbufs[b], acc.at[dstb.at[j]], add=True)
                    if j + NB < IG:
                        pltpu.async_copy(rows_hbm.at[srcb.at[j + NB]],
                                         rowbufs[b], gsems[b])
                return 0

            lax.fori_loop(0, NCHUNK // IG, group, 0)
        else:
            def body(j, _):
                pltpu.sync_copy(rowb0, acc.at[dstb.at[j]], add=True)
                return 0

            lax.fori_loop(0, NCHUNK, body, 0)

        plsc.subcore_barrier()
        pltpu.sync_copy(acc.at[pl.ds(r0, ROWS_PT)],
                        out_hbm.at[c, pl.ds(r0, ROWS_PT)])

    return k


_sc_cache = {}


def _sc_deg(*args):
    if "deg" not in _sc_cache:
        _sc_cache["deg"] = _make_sc_scatter(H, with_gather=False)
    return _sc_cache["deg"](*args)


def _sc_rows(*args):
    if "rows" not in _sc_cache:
        _sc_cache["rows"] = _make_sc_scatter(H, with_gather=True)
    return _sc_cache["rows"](*args)


def _dinv(deg2_blk):
    deg = deg2_blk[0, :, 0:1] + deg2_blk[1, :, 0:1] + 1.0
    return lax.rsqrt(deg)


def _enc_pre1_body(x_ref, wencT_ref, benc_ref, wg1T_ref, deg2_ref, out_ref):
    h0 = jnp.dot(x_ref[...], wencT_ref[...],
                 preferred_element_type=jnp.float32) + benc_ref[...]
    dinv = _dinv(deg2_ref[...])
    out_ref[...] = jnp.dot(h0, wg1T_ref[...],
                           preferred_element_type=jnp.float32) * dinv


def _mid_body(s2_ref, hw_ref, deg2_ref, b_ref, wT_ref, out_ref):
    dinv = _dinv(deg2_ref[...])
    tot = s2_ref[0] + s2_ref[1] + hw_ref[...]
    h1 = jax.nn.relu(dinv * tot + b_ref[...])
    out_ref[...] = jnp.dot(h1, wT_ref[...],
                           preferred_element_type=jnp.float32) * dinv


def _post_body(s2_ref, hw_ref, deg2_ref, b_ref, wihT_ref, bih_ref, out_ref):
    dinv = _dinv(deg2_ref[...])
    tot = s2_ref[0] + s2_ref[1] + hw_ref[...]
    h2 = jax.nn.relu(dinv * tot + b_ref[...])
    out_ref[...] = jnp.dot(h2, wihT_ref[...],
                           preferred_element_type=jnp.float32) + bih_ref[...]


def _gru_body(gi_ref, whhT_ref, bhh_ref, wclsT_ref, bcls_ref,
              ys_ref, logits_ref, carry_ref):
    @pl.when(pl.program_id(0) == 0)
    def _():
        carry_ref[...] = jnp.zeros_like(carry_ref)

    def step(j, hprev):
        gh = jnp.dot(hprev, whhT_ref[...],
                     preferred_element_type=jnp.float32) + bhh_ref[...]
        gi = gi_ref[pl.ds(j, 1), :]
        r = jax.nn.sigmoid(gi[:, 0:H] + gh[:, 0:H])
        z = jax.nn.sigmoid(gi[:, H:2 * H] + gh[:, H:2 * H])
        n = jnp.tanh(gi[:, 2 * H:3 * H] + r * gh[:, 2 * H:3 * H])
        h = (1.0 - z) * n + z * hprev
        ys_ref[pl.ds(j, 1), :] = h
        return h

    def step2(i, hprev):
        return step(2 * i + 1, step(2 * i, hprev))

    h_out = lax.fori_loop(0, GB // 2, step2, carry_ref[0:1, :])
    carry_ref[0:1, :] = h_out
    logits_ref[...] = jnp.dot(ys_ref[...], wclsT_ref[...],
                              preferred_element_type=jnp.float32) + bcls_ref[...]


def _row_spec(width):
    return pl.BlockSpec((BLK, width), lambda i: (i, 0))


def _full_spec(shape):
    return pl.BlockSpec(shape, lambda i: tuple(0 for _ in shape))


def _deg_spec():
    return pl.BlockSpec((2, BLK, H), lambda i: (0, i, 0))


def _tc_rowwise(body, in_specs, out_width, num_outs=1):
    out_shape = jax.ShapeDtypeStruct((NPAD, out_width), jnp.float32)
    return pl.pallas_call(
        body,
        grid=(NPAD // BLK,),
        in_specs=in_specs,
        out_specs=_row_spec(out_width),
        out_shape=out_shape,
    )


def kernel(x, edge_index, timestamps, W_enc, b_enc, W_g1, b_g1, W_g2, b_g2,
           W_ih, W_hh, b_ih, b_hh, W_cls, b_cls):
    del timestamps
    src = edge_index[0]
    dst = edge_index[1]

    xp = jnp.zeros((NPAD, D), jnp.float32).at[:N].set(x)

    npadding = EPAD - E
    pad_idx = jnp.arange(npadding, dtype=jnp.int32)
    srcp = jnp.concatenate(
        [src, pad_idx % N]).reshape(NW * NCHUNK, CH)
    dstp = jnp.concatenate(
        [dst, N + pad_idx % (NPAD - N)]).reshape(NW * NCHUNK, CH)
    z128 = jnp.zeros((NPAD, H), jnp.float32)
    ones128 = jnp.ones((CH, H), jnp.float32)

    deg2 = _sc_deg(ones128, dstp, z128)

    hw1 = _tc_rowwise(
        _enc_pre1_body,
        [_row_spec(D), _full_spec((D, H)), _full_spec((1, H)),
         _full_spec((H, H)), _deg_spec()],
        H,
    )(xp, W_enc.T, b_enc[None, :], W_g1.T, deg2)

    s1_2 = _sc_rows(hw1, srcp, dstp, z128)

    hw2 = _tc_rowwise(
        _mid_body,
        [pl.BlockSpec((2, BLK, H), lambda i: (0, i, 0)), _row_spec(H),
         _deg_spec(), _full_spec((1, H)), _full_spec((H, H))],
        H,
    )(s1_2, hw1, deg2, b_g1[None, :], W_g2.T)

    s2_2 = _sc_rows(hw2, srcp, dstp, z128)

    gi = _tc_rowwise(
        _post_body,
        [pl.BlockSpec((2, BLK, H), lambda i: (0, i, 0)), _row_spec(H),
         _deg_spec(), _full_spec((1, H)), _full_spec((H, 3 * H)),
         _full_spec((1, 3 * H))],
        3 * H,
    )(s2_2, hw2, deg2, b_g2[None, :], W_ih.T, b_ih[None, :])

    wcls_pad = jnp.zeros((H, 128), jnp.float32).at[:, :2].set(W_cls.T)
    bcls_pad = jnp.zeros((1, 128), jnp.float32).at[0, :2].set(b_cls)
    ys, logits_pad = pl.pallas_call(
        _gru_body,
        grid=(NPAD // GB,),
        in_specs=[pl.BlockSpec((GB, 3 * H), lambda i: (i, 0)),
                  _full_spec((H, 3 * H)), _full_spec((1, 3 * H)),
                  _full_spec((H, 128)), _full_spec((1, 128))],
        out_specs=[pl.BlockSpec((GB, H), lambda i: (i, 0)),
                   pl.BlockSpec((GB, 128), lambda i: (i, 0))],
        out_shape=[jax.ShapeDtypeStruct((NPAD, H), jnp.float32),
                   jax.ShapeDtypeStruct((NPAD, 128), jnp.float32)],
        scratch_shapes=[pltpu.VMEM((8, H), jnp.float32)],
    )(gi, W_hh.T, b_hh[None, :], wcls_pad, bcls_pad)

    return (logits_pad[:N, :2], ys[:N])


# speedup vs baseline: 1.0764x; 1.0249x over previous
import functools

import jax
import jax.numpy as jnp
from jax import lax
from jax.experimental import pallas as pl
from jax.experimental.pallas import tpu as pltpu
from jax.experimental.pallas import tpu_sc as plsc

N = 10000
E = 320000
D = 128
H = 128

BLK = 512
NPAD = 10240
GB = 2048

NC = 2
NS = 16
NW = NC * NS
CH = 128
NB = 2
IG = 8
NCHUNK = 80
EPT = NCHUNK * CH
EPAD = NW * EPT
ROWS_PT = NPAD // NS


def _make_sc_scatter(width, with_gather):
    mesh = plsc.VectorSubcoreMesh(core_axis_name="c", subcore_axis_name="s",
                                  num_cores=NC, num_subcores=NS)

    if with_gather:
        scratch = ([pltpu.VMEM((IG, CH), jnp.int32)]
                   + [pltpu.VMEM((IG, CH), jnp.int32)]
                   + [pltpu.VMEM((CH, width), jnp.float32)] * NB
                   + [pltpu.VMEM_SHARED((NPAD, width), jnp.float32)]
                   + [pltpu.SemaphoreType.DMA]
                   + [pltpu.SemaphoreType.DMA] * NB)
    else:
        scratch = [
            pltpu.VMEM((NCHUNK, CH), jnp.int32),
            pltpu.VMEM((CH, width), jnp.float32),
            pltpu.VMEM_SHARED((NPAD, width), jnp.float32),
            pltpu.SemaphoreType.DMA,
        ]

    @functools.partial(
        pl.kernel,
        out_type=jax.ShapeDtypeStruct((NC, NPAD, width), jnp.float32),
        mesh=mesh,
        scratch_types=scratch,
    )
    def k(*refs):
        if with_gather:
            (rows_hbm, src_hbm, dst_hbm, zeros_hbm, out_hbm,
             srcb, dstb) = refs[:7]
            rowbufs = refs[7:7 + NB]
            acc, sem = refs[7 + NB:9 + NB]
            gsems = refs[9 + NB:]
        else:
            (ones_hbm, dst_hbm, zeros_hbm, out_hbm,
             dstb, rowb0, acc, sem) = refs
        c = lax.axis_index("c")
        s = lax.axis_index("s")
        wid = c * NS + s

        r0 = s * ROWS_PT
        pltpu.sync_copy(zeros_hbm.at[pl.ds(r0, ROWS_PT)],
                        acc.at[pl.ds(r0, ROWS_PT)])
        if not with_gather:
            pltpu.sync_copy(dst_hbm.at[pl.ds(wid * NCHUNK, NCHUNK)], dstb)
            pltpu.sync_copy(ones_hbm, rowb0)
        plsc.subcore_barrier()

        if with_gather:
            def group(g, _):
                grow = wid * NCHUNK + g * IG
                pltpu.sync_copy(src_hbm.at[pl.ds(grow, IG)], srcb)
                pltpu.sync_copy(dst_hbm.at[pl.ds(grow, IG)], dstb)
                for b in range(NB):
                    pltpu.async_copy(rows_hbm.at[srcb.at[b]], rowbufs[b],
                                     gsems[b])
                for j in range(IG):
                    b = j % NB
                    pltpu.make_async_copy(rows_hbm.at[srcb.at[j]],
                                          rowbufs[b], gsems[b]).wait()
                    pltpu.sync_copy(rowbufs[b], acc.at[dstb.at[j]], add=True)
                    if j + NB < IG:
                        pltpu.async_copy(rows_hbm.at[srcb.at[j + NB]],
                                         rowbufs[b], gsems[b])
                return 0

            lax.fori_loop(0, NCHUNK // IG, group, 0)
        else:
            def body(j, _):
                pltpu.sync_copy(rowb0, acc.at[dstb.at[j]], add=True)
                return 0

            lax.fori_loop(0, NCHUNK, body, 0)

        plsc.subcore_barrier()
        pltpu.sync_copy(acc.at[pl.ds(r0, ROWS_PT)],
                        out_hbm.at[c, pl.ds(r0, ROWS_PT)])

    return k


_sc_cache = {}


def _sc_deg(*args):
    if "deg" not in _sc_cache:
        _sc_cache["deg"] = _make_sc_scatter(H, with_gather=False)
    return _sc_cache["deg"](*args)


def _sc_rows(*args):
    if "rows" not in _sc_cache:
        _sc_cache["rows"] = _make_sc_scatter(H, with_gather=True)
    return _sc_cache["rows"](*args)


def _dinv(deg2_blk):
    deg = deg2_blk[0, :, 0:1] + deg2_blk[1, :, 0:1] + 1.0
    return lax.rsqrt(deg)


def _enc_pre1_body(x_ref, wencT_ref, benc_ref, wg1T_ref, deg2_ref, out_ref):
    h0 = jnp.dot(x_ref[...], wencT_ref[...],
                 preferred_element_type=jnp.float32) + benc_ref[...]
    dinv = _dinv(deg2_ref[...])
    out_ref[...] = jnp.dot(h0, wg1T_ref[...],
                           preferred_element_type=jnp.float32) * dinv


def _mid_body(s2_ref, hw_ref, deg2_ref, b_ref, wT_ref, out_ref):
    dinv = _dinv(deg2_ref[...])
    tot = s2_ref[0] + s2_ref[1] + hw_ref[...]
    h1 = jax.nn.relu(dinv * tot + b_ref[...])
    out_ref[...] = jnp.dot(h1, wT_ref[...],
                           preferred_element_type=jnp.float32) * dinv


def _post_body(s2_ref, hw_ref, deg2_ref, b_ref, wihT_ref, bih_ref, out_ref):
    dinv = _dinv(deg2_ref[...])
    tot = s2_ref[0] + s2_ref[1] + hw_ref[...]
    h2 = jax.nn.relu(dinv * tot + b_ref[...])
    out_ref[...] = jnp.dot(h2, wihT_ref[...],
                           preferred_element_type=jnp.float32) + bih_ref[...]


def _gru_body(gi_ref, whhT_ref, bhh_ref, wclsT_ref, bcls_ref,
              ys_ref, logits_ref, carry_ref):
    @pl.when(pl.program_id(0) == 0)
    def _():
        carry_ref[...] = jnp.zeros_like(carry_ref)

    def step(j, hprev):
        gh = jnp.dot(hprev, whhT_ref[...],
                     preferred_element_type=jnp.float32) + bhh_ref[...]
        gi = gi_ref[pl.ds(j, 1), :]
        r = jax.nn.sigmoid(gi[:, 0:H] + gh[:, 0:H])
        z = jax.nn.sigmoid(gi[:, H:2 * H] + gh[:, H:2 * H])
        n = jnp.tanh(gi[:, 2 * H:3 * H] + r * gh[:, 2 * H:3 * H])
        h = (1.0 - z) * n + z * hprev
        ys_ref[pl.ds(j, 1), :] = h
        return h

    def step4(i, hprev):
        h = step(4 * i, hprev)
        h = step(4 * i + 1, h)
        h = step(4 * i + 2, h)
        return step(4 * i + 3, h)

    h_out = lax.fori_loop(0, GB // 4, step4, carry_ref[0:1, :])
    carry_ref[0:1, :] = h_out
    logits_ref[...] = jnp.dot(ys_ref[...], wclsT_ref[...],
                              preferred_element_type=jnp.float32) + bcls_ref[...]


def _row_spec(width):
    return pl.BlockSpec((BLK, width), lambda i: (i, 0))


def _full_spec(shape):
    return pl.BlockSpec(shape, lambda i: tuple(0 for _ in shape))


def _deg_spec():
    return pl.BlockSpec((2, BLK, H), lambda i: (0, i, 0))


def _tc_rowwise(body, in_specs, out_width, num_outs=1):
    out_shape = jax.ShapeDtypeStruct((NPAD, out_width), jnp.float32)
    return pl.pallas_call(
        body,
        grid=(NPAD // BLK,),
        in_specs=in_specs,
        out_specs=_row_spec(out_width),
        out_shape=out_shape,
    )


def kernel(x, edge_index, timestamps, W_enc, b_enc, W_g1, b_g1, W_g2, b_g2,
           W_ih, W_hh, b_ih, b_hh, W_cls, b_cls):
    del timestamps
    src = edge_index[0]
    dst = edge_index[1]

    xp = jnp.zeros((NPAD, D), jnp.float32).at[:N].set(x)

    npadding = EPAD - E
    pad_idx = jnp.arange(npadding, dtype=jnp.int32)
    srcp = jnp.concatenate(
        [src, pad_idx % N]).reshape(NW * NCHUNK, CH)
    dstp = jnp.concatenate(
        [dst, N + pad_idx % (NPAD - N)]).reshape(NW * NCHUNK, CH)
    z128 = jnp.zeros((NPAD, H), jnp.float32)
    ones128 = jnp.ones((CH, H), jnp.float32)

    deg2 = _sc_deg(ones128, dstp, z128)

    hw1 = _tc_rowwise(
        _enc_pre1_body,
        [_row_spec(D), _full_spec((D, H)), _full_spec((1, H)),
         _full_spec((H, H)), _deg_spec()],
        H,
    )(xp, W_enc.T, b_enc[None, :], W_g1.T, deg2)

    s1_2 = _sc_rows(hw1, srcp, dstp, z128)

    hw2 = _tc_rowwise(
        _mid_body,
        [pl.BlockSpec((2, BLK, H), lambda i: (0, i, 0)), _row_spec(H),
         _deg_spec(), _full_spec((1, H)), _full_spec((H, H))],
        H,
    )(s1_2, hw1, deg2, b_g1[None, :], W_g2.T)

    s2_2 = _sc_rows(hw2, srcp, dstp, z128)

    gi = _tc_rowwise(
        _post_body,
        [pl.BlockSpec((2, BLK, H), lambda i: (0, i, 0)), _row_spec(H),
         _deg_spec(), _full_spec((1, H)), _full_spec((H, 3 * H)),
         _full_spec((1, 3 * H))],
        3 * H,
    )(s2_2, hw2, deg2, b_g2[None, :], W_ih.T, b_ih[None, :])

    wcls_pad = jnp.zeros((H, 128), jnp.float32).at[:, :2].set(W_cls.T)
    bcls_pad = jnp.zeros((1, 128), jnp.float32).at[0, :2].set(b_cls)
    ys, logits_pad = pl.pallas_call(
        _gru_body,
        grid=(NPAD // GB,),
        in_specs=[pl.BlockSpec((GB, 3 * H), lambda i: (i, 0)),
                  _full_spec((H, 3 * H)), _full_spec((1, 3 * H)),
                  _full_spec((H, 128)), _full_spec((1, 128))],
        out_specs=[pl.BlockSpec((GB, H), lambda i: (i, 0)),
                   pl.BlockSpec((GB, 128), lambda i: (i, 0))],
        out_shape=[jax.ShapeDtypeStruct((NPAD, H), jnp.float32),
                   jax.ShapeDtypeStruct((NPAD, 128), jnp.float32)],
        scratch_shapes=[pltpu.VMEM((8, H), jnp.float32)],
    )(gi, W_hh.T, b_hh[None, :], wcls_pad, bcls_pad)

    return (logits_pad[:N, :2], ys[:N])


# speedup vs baseline: 1.0906x; 1.0131x over previous
import functools

import jax
import jax.numpy as jnp
from jax import lax
from jax.experimental import pallas as pl
from jax.experimental.pallas import tpu as pltpu
from jax.experimental.pallas import tpu_sc as plsc

N = 10000
E = 320000
D = 128
H = 128

BLK = 512
NPAD = 10240
GB = 2048

NC = 2
NS = 16
NW = NC * NS
CH = 128
NB = 2
IG = 8
NCHUNK = 80
EPT = NCHUNK * CH
EPAD = NW * EPT
ROWS_PT = NPAD // NS


def _make_sc_scatter(width, with_gather):
    mesh = plsc.VectorSubcoreMesh(core_axis_name="c", subcore_axis_name="s",
                                  num_cores=NC, num_subcores=NS)

    if with_gather:
        scratch = ([pltpu.VMEM((IG, CH), jnp.int32)]
                   + [pltpu.VMEM((IG, CH), jnp.int32)]
                   + [pltpu.VMEM((CH, width), jnp.float32)] * NB
                   + [pltpu.VMEM_SHARED((NPAD, width), jnp.float32)]
                   + [pltpu.SemaphoreType.DMA]
                   + [pltpu.SemaphoreType.DMA] * NB)
    else:
        scratch = [
            pltpu.VMEM((NCHUNK, CH), jnp.int32),
            pltpu.VMEM((CH, width), jnp.float32),
            pltpu.VMEM_SHARED((NPAD, width), jnp.float32),
            pltpu.SemaphoreType.DMA,
        ]

    @functools.partial(
        pl.kernel,
        out_type=jax.ShapeDtypeStruct((NC, NPAD, width), jnp.float32),
        mesh=mesh,
        scratch_types=scratch,
    )
    def k(*refs):
        if with_gather:
            (rows_hbm, src_hbm, dst_hbm, zeros_hbm, out_hbm,
             srcb, dstb) = refs[:7]
            rowbufs = refs[7:7 + NB]
            acc, sem = refs[7 + NB:9 + NB]
            gsems = refs[9 + NB:]
        else:
            (ones_hbm, dst_hbm, zeros_hbm, out_hbm,
             dstb, rowb0, acc, sem) = refs
        c = lax.axis_index("c")
        s = lax.axis_index("s")
        wid = c * NS + s

        r0 = s * ROWS_PT
        pltpu.sync_copy(zeros_hbm.at[pl.ds(r0, ROWS_PT)],
                        acc.at[pl.ds(r0, ROWS_PT)])
        if not with_gather:
            pltpu.sync_copy(dst_hbm.at[pl.ds(wid * NCHUNK, NCHUNK)], dstb)
            pltpu.sync_copy(ones_hbm, rowb0)
        plsc.subcore_barrier()

        if with_gather:
            def group(g, _):
                grow = wid * NCHUNK + g * IG
                pltpu.sync_copy(src_hbm.at[pl.ds(grow, IG)], srcb)
                pltpu.sync_copy(dst_hbm.at[pl.ds(grow, IG)], dstb)
                for b in range(NB):
                    pltpu.async_copy(rows_hbm.at[srcb.at[b]], rowbufs[b],
                                     gsems[b])
                for j in range(IG):
                    b = j % NB
                    pltpu.make_async_copy(rows_hbm.at[srcb.at[j]],
                                          rowbufs[b], gsems[b]).wait()
                    pltpu.sync_copy(rowbufs[b], acc.at[dstb.at[j]], add=True)
                    if j + NB < IG:
                        pltpu.async_copy(rows_hbm.at[srcb.at[j + NB]],
                                         rowbufs[b], gsems[b])
                return 0

            lax.fori_loop(0, NCHUNK // IG, group, 0)
        else:
            def body(j, _):
                pltpu.sync_copy(rowb0, acc.at[dstb.at[j]], add=True)
                return 0

            lax.fori_loop(0, NCHUNK, body, 0)

        plsc.subcore_barrier()
        pltpu.sync_copy(acc.at[pl.ds(r0, ROWS_PT)],
                        out_hbm.at[c, pl.ds(r0, ROWS_PT)])

    return k


_sc_cache = {}


def _sc_deg(*args):
    if "deg" not in _sc_cache:
        _sc_cache["deg"] = _make_sc_scatter(H, with_gather=False)
    return _sc_cache["deg"](*args)


def _sc_rows(*args):
    if "rows" not in _sc_cache:
        _sc_cache["rows"] = _make_sc_scatter(H, with_gather=True)
    return _sc_cache["rows"](*args)


def _dinv(deg2_blk):
    deg = deg2_blk[0, :, 0:1] + deg2_blk[1, :, 0:1] + 1.0
    return lax.rsqrt(deg)


def _enc_pre1_body(x_ref, wencT_ref, benc_ref, wg1T_ref, deg2_ref, out_ref):
    h0 = jnp.dot(x_ref[...], wencT_ref[...],
                 preferred_element_type=jnp.float32) + benc_ref[...]
    dinv = _dinv(deg2_ref[...])
    out_ref[...] = jnp.dot(h0, wg1T_ref[...],
                           preferred_element_type=jnp.float32) * dinv


def _mid_body(s2_ref, hw_ref, deg2_ref, b_ref, wT_ref, out_ref):
    dinv = _dinv(deg2_ref[...])
    tot = s2_ref[0] + s2_ref[1] + hw_ref[...]
    h1 = jax.nn.relu(dinv * tot + b_ref[...])
    out_ref[...] = jnp.dot(h1, wT_ref[...],
                           preferred_element_type=jnp.float32) * dinv


def _post_body(s2_ref, hw_ref, deg2_ref, b_ref, wihT_ref, bih_ref, out_ref):
    dinv = _dinv(deg2_ref[...])
    tot = s2_ref[0] + s2_ref[1] + hw_ref[...]
    h2 = jax.nn.relu(dinv * tot + b_ref[...])
    out_ref[...] = jnp.dot(h2, wihT_ref[...],
                           preferred_element_type=jnp.float32) + bih_ref[...]


def _gru_body(gi_ref, whhT_ref, bhh_ref, wclsT_ref, bcls_ref,
              ys_ref, logits_ref, carry_ref):
    @pl.when(pl.program_id(0) == 0)
    def _():
        carry_ref[...] = jnp.zeros_like(carry_ref)

    def step(j, hprev):
        gh = jnp.dot(hprev, whhT_ref[...],
                     preferred_element_type=jnp.float32) + bhh_ref[...]
        gi = gi_ref[pl.ds(j, 1), :]
        r = jax.nn.sigmoid(gi[:, 0:H] + gh[:, 0:H])
        z = jax.nn.sigmoid(gi[:, H:2 * H] + gh[:, H:2 * H])
        n = jnp.tanh(gi[:, 2 * H:3 * H] + r * gh[:, 2 * H:3 * H])
        h = (1.0 - z) * n + z * hprev
        ys_ref[pl.ds(j, 1), :] = h
        return h

    def step8(i, hprev):
        h = hprev
        for u in range(8):
            h = step(8 * i + u, h)
        return h

    h_out = lax.fori_loop(0, GB // 8, step8, carry_ref[0:1, :])
    carry_ref[0:1, :] = h_out
    logits_ref[...] = jnp.dot(ys_ref[...], wclsT_ref[...],
                              preferred_element_type=jnp.float32) + bcls_ref[...]


def _row_spec(width):
    return pl.BlockSpec((BLK, width), lambda i: (i, 0))


def _full_spec(shape):
    return pl.BlockSpec(shape, lambda i: tuple(0 for _ in shape))


def _deg_spec():
    return pl.BlockSpec((2, BLK, H), lambda i: (0, i, 0))


def _tc_rowwise(body, in_specs, out_width, num_outs=1):
    out_shape = jax.ShapeDtypeStruct((NPAD, out_width), jnp.float32)
    return pl.pallas_call(
        body,
        grid=(NPAD // BLK,),
        in_specs=in_specs,
        out_specs=_row_spec(out_width),
        out_shape=out_shape,
    )


def kernel(x, edge_index, timestamps, W_enc, b_enc, W_g1, b_g1, W_g2, b_g2,
           W_ih, W_hh, b_ih, b_hh, W_cls, b_cls):
    del timestamps
    src = edge_index[0]
    dst = edge_index[1]

    xp = jnp.zeros((NPAD, D), jnp.float32).at[:N].set(x)

    npadding = EPAD - E
    pad_idx = jnp.arange(npadding, dtype=jnp.int32)
    srcp = jnp.concatenate(
        [src, pad_idx % N]).reshape(NW * NCHUNK, CH)
    dstp = jnp.concatenate(
        [dst, N + pad_idx % (NPAD - N)]).reshape(NW * NCHUNK, CH)
    z128 = jnp.zeros((NPAD, H), jnp.float32)
    ones128 = jnp.ones((CH, H), jnp.float32)

    deg2 = _sc_deg(ones128, dstp, z128)

    hw1 = _tc_rowwise(
        _enc_pre1_body,
        [_row_spec(D), _full_spec((D, H)), _full_spec((1, H)),
         _full_spec((H, H)), _deg_spec()],
        H,
    )(xp, W_enc.T, b_enc[None, :], W_g1.T, deg2)

    s1_2 = _sc_rows(hw1, srcp, dstp, z128)

    hw2 = _tc_rowwise(
        _mid_body,
        [pl.BlockSpec((2, BLK, H), lambda i: (0, i, 0)), _row_spec(H),
         _deg_spec(), _full_spec((1, H)), _full_spec((H, H))],
        H,
    )(s1_2, hw1, deg2, b_g1[None, :], W_g2.T)

    s2_2 = _sc_rows(hw2, srcp, dstp, z128)

    gi = _tc_rowwise(
        _post_body,
        [pl.BlockSpec((2, BLK, H), lambda i: (0, i, 0)), _row_spec(H),
         _deg_spec(), _full_spec((1, H)), _full_spec((H, 3 * H)),
         _full_spec((1, 3 * H))],
        3 * H,
    )(s2_2, hw2, deg2, b_g2[None, :], W_ih.T, b_ih[None, :])

    wcls_pad = jnp.zeros((H, 128), jnp.float32).at[:, :2].set(W_cls.T)
    bcls_pad = jnp.zeros((1, 128), jnp.float32).at[0, :2].set(b_cls)
    ys, logits_pad = pl.pallas_call(
        _gru_body,
        grid=(NPAD // GB,),
        in_specs=[pl.BlockSpec((GB, 3 * H), lambda i: (i, 0)),
                  _full_spec((H, 3 * H)), _full_spec((1, 3 * H)),
                  _full_spec((H, 128)), _full_spec((1, 128))],
        out_specs=[pl.BlockSpec((GB, H), lambda i: (i, 0)),
                   pl.BlockSpec((GB, 128), lambda i: (i, 0))],
        out_shape=[jax.ShapeDtypeStruct((NPAD, H), jnp.float32),
                   jax.ShapeDtypeStruct((NPAD, 128), jnp.float32)],
        scratch_shapes=[pltpu.VMEM((8, H), jnp.float32)],
    )(gi, W_hh.T, b_hh[None, :], wcls_pad, bcls_pad)

    return (logits_pad[:N, :2], ys[:N])
